# baseline (speedup 1.0000x reference)
import functools
import math

import jax
import jax.numpy as jnp
from jax import lax
from jax.experimental import pallas as pl
from jax.experimental.pallas import tpu as pltpu
from jax.experimental.pallas import tpu_sc as plsc

_V = 100000
_N = 1024
_PAD = 0
_SMOOTH = 0.1
_CONF = 1.0 - _SMOOTH
_EPS = _SMOOTH / (_V - 2)
_C0 = (_V - 2) * _EPS * math.log(_EPS) + _CONF * math.log(_CONF)

_NC = 2
_NS = 16
_NW = _NC * _NS
_BPW = _N // _NW

_VSC = 28672
_VTC = _V - _VSC
_RSC = _VSC // _NW
_CH = 32
_NCH = _RSC // _CH
_LC = _N // 16

_RPB = 3584
_NB = (_VTC + _RPB - 1) // _RPB


@functools.cache
def _sc_kernel():
    @functools.partial(
        pl.kernel,
        mesh=plsc.VectorSubcoreMesh(core_axis_name="c", subcore_axis_name="s"),
        out_type=(
            jax.ShapeDtypeStruct((_N,), jnp.float32),
            jax.ShapeDtypeStruct((_NW, 3, _N), jnp.float32),
        ),
        scratch_types=[
            pltpu.VMEM((_BPW,), jnp.int32),
            pltpu.VMEM((_BPW, _N), jnp.float32),
            pltpu.VMEM((_BPW,), jnp.float32),
            pltpu.VMEM((_CH, _N), jnp.float32),
            pltpu.VMEM((_CH, _N), jnp.float32),
            pltpu.VMEM((3, _N), jnp.float32),
            pltpu.SemaphoreType.DMA,
            pltpu.SemaphoreType.DMA,
            pltpu.SemaphoreType.DMA,
        ],
    )
    def _sc_body(predt_hbm, tgt_hbm, pt_hbm, stats_hbm, tgt_v, vals_v, out_v,
                 buf0, buf1, acc, gsem, sem0, sem1):
        wid = lax.axis_index("s") * _NC + lax.axis_index("c")
        base = wid * _BPW

        pltpu.sync_copy(tgt_hbm.at[pl.ds(base, _BPW)], tgt_v)
        pltpu.async_copy(predt_hbm.at[tgt_v], vals_v, gsem).wait()
        io = lax.iota(jnp.int32, 16)
        for k in range(_BPW // 16):
            lane0 = base + 16 * k
            sel = jnp.zeros((16,), jnp.float32)
            for i in range(16):
                row = vals_v[16 * k + i, pl.ds(lane0, 16)]
                sel = jnp.where(io == i, row, sel)
            out_v[pl.ds(16 * k, 16)] = sel
        pltpu.sync_copy(out_v, pt_hbm.at[pl.ds(base, _BPW)])

        def _initacc(c, _):
            c16 = c * 16
            acc[0, pl.ds(c16, 16)] = jnp.full((16,), -jnp.inf, jnp.float32)
            acc[1, pl.ds(c16, 16)] = jnp.zeros((16,), jnp.float32)
            acc[2, pl.ds(c16, 16)] = jnp.zeros((16,), jnp.float32)
            return _

        lax.fori_loop(0, _LC, _initacc, None)

        row0 = _VTC + wid * _RSC
        bufs = (buf0, buf1)
        sems = (sem0, sem1)

        pltpu.async_copy(predt_hbm.at[pl.ds(row0, _CH)], buf0, sem0)
        pltpu.async_copy(predt_hbm.at[pl.ds(row0 + _CH, _CH)], buf1, sem1)

        def _reduce_chunk(buf):
            def _lanes(c, _):
                c16 = c * 16
                m = acc[0, pl.ds(c16, 16)]
                s = acc[1, pl.ds(c16, 16)]
                sv = acc[2, pl.ds(c16, 16)]
                xs = [buf[r, pl.ds(c16, 16)] for r in range(_CH)]
                t = xs
                while len(t) > 1:
                    t = [jnp.maximum(t[i], t[i + 1])
                         for i in range(0, len(t) - 1, 2)] + (
                             [t[-1]] if len(t) % 2 else [])
                bm = jnp.maximum(m, t[0])
                es = [jnp.exp(x - bm) for x in xs]
                while len(es) > 1:
                    es = [es[i] + es[i + 1]
                          for i in range(0, len(es) - 1, 2)] + (
                              [es[-1]] if len(es) % 2 else [])
                vs = xs
                while len(vs) > 1:
                    vs = [vs[i] + vs[i + 1]
                          for i in range(0, len(vs) - 1, 2)] + (
                              [vs[-1]] if len(vs) % 2 else [])
                acc[0, pl.ds(c16, 16)] = bm
                acc[1, pl.ds(c16, 16)] = s * jnp.exp(m - bm) + es[0]
                acc[2, pl.ds(c16, 16)] = sv + vs[0]
                return _

            lax.fori_loop(0, _LC, _lanes, None)

        @pl.loop(0, _NCH, step=2)
        def _chunks(ch):
            for b in range(2):
                pltpu.make_async_copy(
                    predt_hbm.at[pl.ds(row0, _CH)], bufs[b], sems[b]
                ).wait()
                _reduce_chunk(bufs[b])
                nxt = ch + b + 2

                @pl.when(nxt < _NCH)
                def _fire():
                    pltpu.async_copy(
                        predt_hbm.at[pl.ds(row0 + nxt * _CH, _CH)],
                        bufs[b],
                        sems[b],
                    )

        pltpu.sync_copy(acc, stats_hbm.at[wid])

    return _sc_body


def _tc_stats_body(predt_ref, out_ref, m_ref, s_ref, sv_ref):
    j = pl.program_id(0)

    @pl.when(j == 0)
    def _init():
        m_ref[:] = jnp.full((1, _N), -jnp.inf, jnp.float32)
        s_ref[:] = jnp.zeros((1, _N), jnp.float32)
        sv_ref[:] = jnp.zeros((1, _N), jnp.float32)

    x = predt_ref[:]
    m_old = m_ref[:]

    @pl.when(j < _NB - 1)
    def _full():
        bmax = jnp.max(x, axis=0, keepdims=True)
        m_new = jnp.maximum(m_old, bmax)
        s_ref[:] = s_ref[:] * jnp.exp(m_old - m_new) + jnp.sum(
            jnp.exp(x - m_new), axis=0, keepdims=True
        )
        m_ref[:] = m_new
        sv_ref[:] = sv_ref[:] + jnp.sum(x, axis=0, keepdims=True)

    @pl.when(j == _NB - 1)
    def _tail():
        rows = j * _RPB + lax.broadcasted_iota(jnp.int32, (_RPB, _N), 0)
        valid = rows < _VTC
        xm = jnp.where(valid, x, -jnp.inf)
        bmax = jnp.max(xm, axis=0, keepdims=True)
        m_new = jnp.maximum(m_old, bmax)
        s = s_ref[:] * jnp.exp(m_old - m_new) + jnp.sum(
            jnp.exp(xm - m_new), axis=0, keepdims=True
        )
        sv = sv_ref[:] + jnp.sum(jnp.where(valid, x, 0.0), axis=0,
                                 keepdims=True)
        out_ref[0:1, :] = m_new
        out_ref[1:2, :] = s
        out_ref[2:3, :] = sv

    @pl.when(j == 0)
    def _p0():
        out_ref[3:4, :] = predt_ref[0:1, :]


def _combine_body(tcs_ref, scs_ref, pt_ref, tgt_ref, out_ref):
    m_tc = tcs_ref[0:1, :]
    s_tc = tcs_ref[1:2, :]
    sv_tc = tcs_ref[2:3, :]
    p0 = tcs_ref[3:4, :]
    sc = scs_ref[:]
    m_sc = sc[:, 0, :]
    s_sc = sc[:, 1, :]
    sv_sc = sc[:, 2, :]
    m_all = jnp.maximum(jnp.max(m_sc, axis=0, keepdims=True), m_tc)
    s_all = s_tc * jnp.exp(m_tc - m_all) + jnp.sum(
        s_sc * jnp.exp(m_sc - m_all), axis=0, keepdims=True
    )
    sv_all = sv_tc + jnp.sum(sv_sc, axis=0, keepdims=True)
    lse = m_all + jnp.log(s_all)
    s_logp = sv_all - jnp.float32(_V) * lse
    logp0 = p0 - lse
    logpt = pt_ref[:] - lse
    row = _C0 - (_EPS * (s_logp - logp0) + (_CONF - _EPS) * logpt)
    row = jnp.where(tgt_ref[:] != _PAD, row, 0.0)
    out_ref[:] = (jnp.sum(row) / _N).reshape(1, 1)


def kernel(pred, target):
    tgt = target.astype(jnp.int32)
    predt = pred.T
    pt, sc_stats = _sc_kernel()(predt, tgt)
    tc_stats = pl.pallas_call(
        _tc_stats_body,
        grid=(_NB,),
        in_specs=[pl.BlockSpec((_RPB, _N), lambda j: (j, 0))],
        out_specs=pl.BlockSpec((4, _N), lambda j: (0, 0)),
        out_shape=jax.ShapeDtypeStruct((4, _N), jnp.float32),
        scratch_shapes=[pltpu.VMEM((1, _N), jnp.float32)] * 3,
        compiler_params=pltpu.CompilerParams(
            dimension_semantics=("arbitrary",)
        ),
    )(predt)
    out = pl.pallas_call(
        _combine_body,
        out_shape=jax.ShapeDtypeStruct((1, 1), jnp.float32),
    )(tc_stats, sc_stats, pt.reshape(1, _N), tgt.reshape(1, _N))
    return out[0, 0]


# speedup vs baseline: 1.0269x; 1.0269x over previous
import functools
import math

import jax
import jax.numpy as jnp
from jax import lax
from jax.experimental import pallas as pl
from jax.experimental.pallas import tpu as pltpu
from jax.experimental.pallas import tpu_sc as plsc

_V = 100000
_N = 1024
_PAD = 0
_SMOOTH = 0.1
_CONF = 1.0 - _SMOOTH
_EPS = _SMOOTH / (_V - 2)
_C0 = (_V - 2) * _EPS * math.log(_EPS) + _CONF * math.log(_CONF)

_NC = 2
_NS = 16
_NW = _NC * _NS
_BPW = _N // _NW

_VSC = 26624
_VTC = _V - _VSC
_RSC = _VSC // _NW
_CH = 32
_NCH = _RSC // _CH
_LC = _N // 16

_RPB = 3584
_NB = (_VTC + _RPB - 1) // _RPB


@functools.cache
def _sc_kernel():
    @functools.partial(
        pl.kernel,
        mesh=plsc.VectorSubcoreMesh(core_axis_name="c", subcore_axis_name="s"),
        out_type=(
            jax.ShapeDtypeStruct((_N,), jnp.float32),
            jax.ShapeDtypeStruct((_NW, 3, _N), jnp.float32),
        ),
        scratch_types=[
            pltpu.VMEM((_BPW,), jnp.int32),
            pltpu.VMEM((_BPW, _N), jnp.float32),
            pltpu.VMEM((_BPW,), jnp.float32),
            pltpu.VMEM((_CH, _N), jnp.float32),
            pltpu.VMEM((_CH, _N), jnp.float32),
            pltpu.VMEM((3, _N), jnp.float32),
            pltpu.SemaphoreType.DMA,
            pltpu.SemaphoreType.DMA,
            pltpu.SemaphoreType.DMA,
        ],
    )
    def _sc_body(predt_hbm, tgt_hbm, pt_hbm, stats_hbm, tgt_v, vals_v, out_v,
                 buf0, buf1, acc, gsem, sem0, sem1):
        wid = lax.axis_index("s") * _NC + lax.axis_index("c")
        base = wid * _BPW

        pltpu.sync_copy(tgt_hbm.at[pl.ds(base, _BPW)], tgt_v)
        pltpu.async_copy(predt_hbm.at[tgt_v], vals_v, gsem).wait()
        io = lax.iota(jnp.int32, 16)
        for k in range(_BPW // 16):
            lane0 = base + 16 * k
            sel = jnp.zeros((16,), jnp.float32)
            for i in range(16):
                row = vals_v[16 * k + i, pl.ds(lane0, 16)]
                sel = jnp.where(io == i, row, sel)
            out_v[pl.ds(16 * k, 16)] = sel
        pltpu.sync_copy(out_v, pt_hbm.at[pl.ds(base, _BPW)])

        def _initacc(c, _):
            c16 = c * 16
            acc[0, pl.ds(c16, 16)] = jnp.full((16,), -jnp.inf, jnp.float32)
            acc[1, pl.ds(c16, 16)] = jnp.zeros((16,), jnp.float32)
            acc[2, pl.ds(c16, 16)] = jnp.zeros((16,), jnp.float32)
            return _

        lax.fori_loop(0, _LC, _initacc, None)

        row0 = _VTC + wid * _RSC
        bufs = (buf0, buf1)
        sems = (sem0, sem1)

        pltpu.async_copy(predt_hbm.at[pl.ds(row0, _CH)], buf0, sem0)
        pltpu.async_copy(predt_hbm.at[pl.ds(row0 + _CH, _CH)], buf1, sem1)

        def _reduce_chunk(buf):
            def _lanes(c, _):
                c16 = c * 16
                m = acc[0, pl.ds(c16, 16)]
                s = acc[1, pl.ds(c16, 16)]
                sv = acc[2, pl.ds(c16, 16)]
                xs = [buf[r, pl.ds(c16, 16)] for r in range(_CH)]
                t = xs
                while len(t) > 1:
                    t = [jnp.maximum(t[i], t[i + 1])
                         for i in range(0, len(t) - 1, 2)] + (
                             [t[-1]] if len(t) % 2 else [])
                bm = jnp.maximum(m, t[0])
                es = [jnp.exp(x - bm) for x in xs]
                while len(es) > 1:
                    es = [es[i] + es[i + 1]
                          for i in range(0, len(es) - 1, 2)] + (
                              [es[-1]] if len(es) % 2 else [])
                vs = xs
                while len(vs) > 1:
                    vs = [vs[i] + vs[i + 1]
                          for i in range(0, len(vs) - 1, 2)] + (
                              [vs[-1]] if len(vs) % 2 else [])
                acc[0, pl.ds(c16, 16)] = bm
                acc[1, pl.ds(c16, 16)] = s * jnp.exp(m - bm) + es[0]
                acc[2, pl.ds(c16, 16)] = sv + vs[0]
                return _

            lax.fori_loop(0, _LC, _lanes, None)

        @pl.loop(0, _NCH, step=2)
        def _chunks(ch):
            for b in range(2):
                pltpu.make_async_copy(
                    predt_hbm.at[pl.ds(row0, _CH)], bufs[b], sems[b]
                ).wait()
                _reduce_chunk(bufs[b])
                nxt = ch + b + 2

                @pl.when(nxt < _NCH)
                def _fire():
                    pltpu.async_copy(
                        predt_hbm.at[pl.ds(row0 + nxt * _CH, _CH)],
                        bufs[b],
                        sems[b],
                    )

        pltpu.sync_copy(acc, stats_hbm.at[wid])

    return _sc_body


def _tc_stats_body(predt_ref, out_ref, m_ref, s_ref, sv_ref):
    j = pl.program_id(0)

    @pl.when(j == 0)
    def _init():
        m_ref[:] = jnp.full((1, _N), -jnp.inf, jnp.float32)
        s_ref[:] = jnp.zeros((1, _N), jnp.float32)
        sv_ref[:] = jnp.zeros((1, _N), jnp.float32)

    x = predt_ref[:]
    m_old = m_ref[:]

    @pl.when(j < _NB - 1)
    def _full():
        bmax = jnp.max(x, axis=0, keepdims=True)
        m_new = jnp.maximum(m_old, bmax)
        s_ref[:] = s_ref[:] * jnp.exp(m_old - m_new) + jnp.sum(
            jnp.exp(x - m_new), axis=0, keepdims=True
        )
        m_ref[:] = m_new
        sv_ref[:] = sv_ref[:] + jnp.sum(x, axis=0, keepdims=True)

    @pl.when(j == _NB - 1)
    def _tail():
        rows = j * _RPB + lax.broadcasted_iota(jnp.int32, (_RPB, _N), 0)
        valid = rows < _VTC
        xm = jnp.where(valid, x, -jnp.inf)
        bmax = jnp.max(xm, axis=0, keepdims=True)
        m_new = jnp.maximum(m_old, bmax)
        s = s_ref[:] * jnp.exp(m_old - m_new) + jnp.sum(
            jnp.exp(xm - m_new), axis=0, keepdims=True
        )
        sv = sv_ref[:] + jnp.sum(jnp.where(valid, x, 0.0), axis=0,
                                 keepdims=True)
        out_ref[0:1, :] = m_new
        out_ref[1:2, :] = s
        out_ref[2:3, :] = sv

    @pl.when(j == 0)
    def _p0():
        out_ref[3:4, :] = predt_ref[0:1, :]


def _combine_body(tcs_ref, scs_ref, pt_ref, tgt_ref, out_ref):
    m_tc = tcs_ref[0:1, :]
    s_tc = tcs_ref[1:2, :]
    sv_tc = tcs_ref[2:3, :]
    p0 = tcs_ref[3:4, :]
    sc = scs_ref[:]
    m_sc = sc[:, 0, :]
    s_sc = sc[:, 1, :]
    sv_sc = sc[:, 2, :]
    m_all = jnp.maximum(jnp.max(m_sc, axis=0, keepdims=True), m_tc)
    s_all = s_tc * jnp.exp(m_tc - m_all) + jnp.sum(
        s_sc * jnp.exp(m_sc - m_all), axis=0, keepdims=True
    )
    sv_all = sv_tc + jnp.sum(sv_sc, axis=0, keepdims=True)
    lse = m_all + jnp.log(s_all)
    s_logp = sv_all - jnp.float32(_V) * lse
    logp0 = p0 - lse
    logpt = pt_ref[:] - lse
    row = _C0 - (_EPS * (s_logp - logp0) + (_CONF - _EPS) * logpt)
    row = jnp.where(tgt_ref[:] != _PAD, row, 0.0)
    out_ref[:] = (jnp.sum(row) / _N).reshape(1, 1)


def kernel(pred, target):
    tgt = target.astype(jnp.int32)
    predt = pred.T
    pt, sc_stats = _sc_kernel()(predt, tgt)
    tc_stats = pl.pallas_call(
        _tc_stats_body,
        grid=(_NB,),
        in_specs=[pl.BlockSpec((_RPB, _N), lambda j: (j, 0))],
        out_specs=pl.BlockSpec((4, _N), lambda j: (0, 0)),
        out_shape=jax.ShapeDtypeStruct((4, _N), jnp.float32),
        scratch_shapes=[pltpu.VMEM((1, _N), jnp.float32)] * 3,
        compiler_params=pltpu.CompilerParams(
            dimension_semantics=("arbitrary",)
        ),
    )(predt)
    out = pl.pallas_call(
        _combine_body,
        out_shape=jax.ShapeDtypeStruct((1, 1), jnp.float32),
    )(tc_stats, sc_stats, pt.reshape(1, _N), tgt.reshape(1, _N))
    return out[0, 0]
